# Initial kernel scaffold; baseline (speedup 1.0000x reference)
import jax
import jax.numpy as jnp
from jax.experimental import pallas as pl


def kernel(x, edge_index, edge_attr, Wn1, Ws1, b1, Wn2, Ws2, b2, W3, b3, W4, b4):
    raise NotImplementedError("write your pallas kernel here")


# speedup vs baseline: 2.2000x; 2.2000x over previous
import functools

import jax
import jax.numpy as jnp
from jax import lax
from jax.experimental import pallas as pl
from jax.experimental.pallas import tpu as pltpu
from jax.experimental.pallas import tpu_sc as plsc

N_NODES = 10000
N_EDGES = 320000
NPAD = 10240
EPAD = 327680
D_NODE = 128
D_EDGE = 16
HID = 128

NCORES = 2
NSUB = 16
NTILES = NCORES * NSUB
CHUNK = 128
CH_PER_TILE = EPAD // NTILES // CHUNK
ROWS_PER_TILE = NPAD // NSUB

_DOT = functools.partial(
    lax.dot_general,
    precision=lax.Precision.HIGHEST,
    preferred_element_type=jnp.float32,
)


def _mm(a, b):
    return _DOT(a, b, (((1,), (0,)), ((), ())))


_sc_mesh = plsc.VectorSubcoreMesh(core_axis_name="c", subcore_axis_name="s")

_NBUF = 2
_HALF = CH_PER_TILE // 2


@functools.partial(
    pl.kernel,
    out_type=jax.ShapeDtypeStruct((NCORES, NPAD, HID), jnp.float32),
    mesh=_sc_mesh,
    scratch_types=[
        pltpu.VMEM((_HALF, CHUNK), jnp.int32),
        pltpu.VMEM((_HALF, CHUNK), jnp.int32),
        pltpu.VMEM((_NBUF, CHUNK, HID), jnp.float32),
        pltpu.VMEM_SHARED((NPAD, HID), jnp.float32),
        pltpu.SemaphoreType.DMA((_NBUF,)),
        pltpu.SemaphoreType.DMA((_NBUF,)),
    ],
)
def _sc_agg(table, srcm, dstm, zeros_hbm, parts, src_v, dst_v, rows_v, acc,
            gsem, ssem):
    c = lax.axis_index("c")
    s = lax.axis_index("s")
    wid = c * NSUB + s

    for i in range(ROWS_PER_TILE // CHUNK):
        pltpu.sync_copy(
            zeros_hbm, acc.at[pl.ds(s * ROWS_PER_TILE + i * CHUNK, CHUNK)]
        )
    plsc.subcore_barrier()

    for half in range(2):
        base_ch = wid * CH_PER_TILE + half * _HALF
        pltpu.sync_copy(srcm.at[pl.ds(base_ch, _HALF)], src_v)
        pltpu.sync_copy(dstm.at[pl.ds(base_ch, _HALF)], dst_v)

        for b in range(_NBUF):
            pltpu.async_copy(table.at[src_v.at[b]], rows_v.at[b], gsem.at[b])

        def group(jj, carry):
            base = jj * _NBUF
            for b in range(_NBUF):
                j = base + b
                pltpu.make_async_copy(
                    table.at[src_v.at[j]], rows_v.at[b], gsem.at[b]
                ).wait()
                pltpu.async_copy(
                    rows_v.at[b], acc.at[dst_v.at[j]], ssem.at[b], add=True
                )
            for b in range(_NBUF):
                j = base + b
                jn = j + _NBUF
                pltpu.make_async_copy(
                    rows_v.at[b], acc.at[dst_v.at[j]], ssem.at[b]
                ).wait()

                @pl.when(jn < _HALF)
                def _():
                    pltpu.async_copy(table.at[src_v.at[jn]], rows_v.at[b],
                                     gsem.at[b])

            return carry

        lax.fori_loop(0, _HALF // _NBUF, group, 0)

    plsc.subcore_barrier()

    pltpu.sync_copy(
        acc.at[pl.ds(s * ROWS_PER_TILE, ROWS_PER_TILE)],
        parts.at[c, pl.ds(s * ROWS_PER_TILE, ROWS_PER_TILE)],
    )


_CGRP = 8


@functools.partial(
    pl.kernel,
    out_type=jax.ShapeDtypeStruct((NCORES, NPAD, HID), jnp.float32),
    mesh=_sc_mesh,
    scratch_types=[
        pltpu.VMEM((CH_PER_TILE, CHUNK), jnp.int32),
        pltpu.VMEM((CHUNK, HID), jnp.float32),
        pltpu.VMEM_SHARED((NPAD, HID), jnp.float32),
        pltpu.SemaphoreType.DMA,
    ],
)
def _sc_count(dstm, ones_hbm, zeros_hbm, parts, dst_v, ones_v, acc, sem):
    c = lax.axis_index("c")
    s = lax.axis_index("s")
    wid = c * NSUB + s

    for i in range(ROWS_PER_TILE // CHUNK):
        pltpu.sync_copy(
            zeros_hbm, acc.at[pl.ds(s * ROWS_PER_TILE + i * CHUNK, CHUNK)]
        )
    plsc.subcore_barrier()
    pltpu.sync_copy(ones_hbm, ones_v)
    pltpu.sync_copy(dstm.at[pl.ds(wid * CH_PER_TILE, CH_PER_TILE)], dst_v)

    def group(jj, carry):
        base = jj * _CGRP
        for b in range(_CGRP):
            pltpu.async_copy(ones_v, acc.at[dst_v.at[base + b]], sem,
                             add=True)
        for b in range(_CGRP):
            pltpu.make_async_copy(ones_v, acc.at[dst_v.at[base + b]],
                                  sem).wait()
        return carry

    lax.fori_loop(0, CH_PER_TILE // _CGRP, group, 0)
    plsc.subcore_barrier()

    pltpu.sync_copy(
        acc.at[pl.ds(s * ROWS_PER_TILE, ROWS_PER_TILE)],
        parts.at[c, pl.ds(s * ROWS_PER_TILE, ROWS_PER_TILE)],
    )


@functools.partial(
    pl.kernel,
    out_type=jax.ShapeDtypeStruct((EPAD, HID), jnp.float32),
    mesh=_sc_mesh,
    scratch_types=[
        pltpu.VMEM((CH_PER_TILE, CHUNK), jnp.int32),
        pltpu.VMEM((CH_PER_TILE, CHUNK), jnp.int32),
        pltpu.VMEM((2, CHUNK, HID), jnp.float32),
        pltpu.VMEM((2, CHUNK, HID), jnp.float32),
        pltpu.SemaphoreType.DMA((2,)),
        pltpu.SemaphoreType.DMA((2,)),
        pltpu.SemaphoreType.DMA((2,)),
    ],
)
def _sc_edge_gather(ptab, qtab, srcm, dstm, s_out, src_v, dst_v, bufp, bufq,
                    psem, qsem, wsem):
    c = lax.axis_index("c")
    s = lax.axis_index("s")
    wid = c * NSUB + s
    ebase = wid * CH_PER_TILE * CHUNK

    pltpu.sync_copy(srcm.at[pl.ds(wid * CH_PER_TILE, CH_PER_TILE)], src_v)
    pltpu.sync_copy(dstm.at[pl.ds(wid * CH_PER_TILE, CH_PER_TILE)], dst_v)

    for b in range(2):
        pltpu.async_copy(ptab.at[src_v.at[b]], bufp.at[b], psem.at[b])
        pltpu.async_copy(qtab.at[dst_v.at[b]], bufq.at[b], qsem.at[b])

    def group(jj, carry):
        for b in range(2):
            j = jj * 2 + b
            jn = j + 2
            pltpu.make_async_copy(ptab.at[src_v.at[j]], bufp.at[b],
                                  psem.at[b]).wait()
            pltpu.make_async_copy(qtab.at[dst_v.at[j]], bufq.at[b],
                                  qsem.at[b]).wait()

            def add_row(r, inner):
                for k in range(HID // 16):
                    sl = pl.ds(k * 16, 16)
                    bufp[b, r, sl] = bufp[b, r, sl] + bufq[b, r, sl]
                return inner

            lax.fori_loop(0, CHUNK, add_row, 0)
            pltpu.async_copy(
                bufp.at[b], s_out.at[pl.ds(ebase + j * CHUNK, CHUNK)],
                wsem.at[b]
            )

            @pl.when(jn < CH_PER_TILE)
            def _():
                pltpu.make_async_copy(
                    bufp.at[b], s_out.at[pl.ds(ebase + j * CHUNK, CHUNK)],
                    wsem.at[b]
                ).wait()
                pltpu.async_copy(ptab.at[src_v.at[jn]], bufp.at[b],
                                 psem.at[b])
                pltpu.async_copy(qtab.at[dst_v.at[jn]], bufq.at[b],
                                 qsem.at[b])

        return carry

    lax.fori_loop(0, CH_PER_TILE // 2, group, 0)
    for j in (CH_PER_TILE - 2, CH_PER_TILE - 1):
        b = j % 2
        pltpu.make_async_copy(
            bufp.at[b], s_out.at[pl.ds(ebase + j * CHUNK, CHUNK)], wsem.at[b]
        ).wait()


_NODE_BLK = 1024
_NODE_GRID = NPAD // _NODE_BLK


def _tc_layer1(parts, cparts, x, wn, ws, b):
    def kern(p_ref, c_ref, x_ref, wn_ref, ws_ref, b_ref, h_ref, inv_ref):
        cnt = (c_ref[0] + c_ref[1])[:, 0:1]
        inv = 1.0 / jnp.maximum(cnt, 1.0)
        mean = (p_ref[0] + p_ref[1]) * inv
        h = _mm(mean, wn_ref[...]) + _mm(x_ref[...], ws_ref[...]) + b_ref[...]
        h_ref[...] = jnp.maximum(h, 0.0)
        inv_ref[...] = inv

    return pl.pallas_call(
        kern,
        grid=(_NODE_GRID,),
        in_specs=[
            pl.BlockSpec((NCORES, _NODE_BLK, HID), lambda i: (0, i, 0)),
            pl.BlockSpec((NCORES, _NODE_BLK, HID), lambda i: (0, i, 0)),
            pl.BlockSpec((_NODE_BLK, D_NODE), lambda i: (i, 0)),
            pl.BlockSpec((D_NODE, HID), lambda i: (0, 0)),
            pl.BlockSpec((D_NODE, HID), lambda i: (0, 0)),
            pl.BlockSpec((1, HID), lambda i: (0, 0)),
        ],
        out_specs=[
            pl.BlockSpec((_NODE_BLK, HID), lambda i: (i, 0)),
            pl.BlockSpec((_NODE_BLK, 1), lambda i: (i, 0)),
        ],
        out_shape=[
            jax.ShapeDtypeStruct((NPAD, HID), jnp.float32),
            jax.ShapeDtypeStruct((NPAD, 1), jnp.float32),
        ],
    )(parts, cparts, x, wn, ws, b)


def _tc_layer2(parts, inv, h1, wn, ws, b, w3a, w3b, b3):
    def kern(p_ref, inv_ref, h1_ref, wn_ref, ws_ref, b_ref, w3a_ref, w3b_ref,
             b3_ref, p_out, q_out):
        mean = (p_ref[0] + p_ref[1]) * inv_ref[...]
        h2 = _mm(mean, wn_ref[...]) + _mm(h1_ref[...], ws_ref[...]) + b_ref[...]
        h2 = jnp.maximum(h2, 0.0)
        p_out[...] = _mm(h2, w3a_ref[...]) + b3_ref[...]
        q_out[...] = _mm(h2, w3b_ref[...])

    return pl.pallas_call(
        kern,
        grid=(_NODE_GRID,),
        in_specs=[
            pl.BlockSpec((NCORES, _NODE_BLK, HID), lambda i: (0, i, 0)),
            pl.BlockSpec((_NODE_BLK, 1), lambda i: (i, 0)),
            pl.BlockSpec((_NODE_BLK, HID), lambda i: (i, 0)),
            pl.BlockSpec((HID, HID), lambda i: (0, 0)),
            pl.BlockSpec((HID, HID), lambda i: (0, 0)),
            pl.BlockSpec((1, HID), lambda i: (0, 0)),
            pl.BlockSpec((HID, HID), lambda i: (0, 0)),
            pl.BlockSpec((HID, HID), lambda i: (0, 0)),
            pl.BlockSpec((1, HID), lambda i: (0, 0)),
        ],
        out_specs=[
            pl.BlockSpec((_NODE_BLK, HID), lambda i: (i, 0)),
            pl.BlockSpec((_NODE_BLK, HID), lambda i: (i, 0)),
        ],
        out_shape=[
            jax.ShapeDtypeStruct((NPAD, HID), jnp.float32),
            jax.ShapeDtypeStruct((NPAD, HID), jnp.float32),
        ],
    )(parts, inv, h1, wn, ws, b, w3a, w3b, b3)


_EDGE_BLK = 1000
_EDGE_GRID = N_EDGES // _EDGE_BLK


def _tc_edge_mlp(s_arr, attr, w3c, w4, b4):
    def kern(s_ref, a_ref, w3c_ref, w4_ref, b4_ref, o_ref):
        u = s_ref[...] + _mm(a_ref[...], w3c_ref[...])
        u = jnp.maximum(u, 0.0)
        o_ref[...] = _mm(u, w4_ref[...]) + b4_ref[...]

    return pl.pallas_call(
        kern,
        grid=(_EDGE_GRID,),
        in_specs=[
            pl.BlockSpec((_EDGE_BLK, HID), lambda i: (i, 0)),
            pl.BlockSpec((_EDGE_BLK, D_EDGE), lambda i: (i, 0)),
            pl.BlockSpec((D_EDGE, HID), lambda i: (0, 0)),
            pl.BlockSpec((HID, 1), lambda i: (0, 0)),
            pl.BlockSpec((1, 1), lambda i: (0, 0)),
        ],
        out_specs=pl.BlockSpec((_EDGE_BLK, 1), lambda i: (i, 0)),
        out_shape=jax.ShapeDtypeStruct((N_EDGES, 1), jnp.float32),
    )(s_arr, attr, w3c, w4, b4)


def kernel(x, edge_index, edge_attr, Wn1, Ws1, b1, Wn2, Ws2, b2, W3, b3, W4, b4):
    f32 = jnp.float32

    src = edge_index[0].astype(jnp.int32)
    dst = edge_index[1].astype(jnp.int32)
    pad = jnp.full((EPAD - N_EDGES,), N_NODES, jnp.int32)
    srcm = jnp.concatenate([src, pad]).reshape(EPAD // CHUNK, CHUNK)
    dstm = jnp.concatenate([dst, pad]).reshape(EPAD // CHUNK, CHUNK)

    xpad = jnp.zeros((NPAD, D_NODE), f32).at[:N_NODES].set(x.astype(f32))
    zeros_blk = jnp.zeros((CHUNK, HID), f32)
    ones_blk = jnp.ones((CHUNK, HID), f32)

    cparts = _sc_count(dstm, ones_blk, zeros_blk)

    parts1 = _sc_agg(xpad, srcm, dstm, zeros_blk)
    h1, inv = _tc_layer1(parts1, cparts, xpad, Wn1, Ws1, b1.reshape(1, HID))

    parts2 = _sc_agg(h1, srcm, dstm, zeros_blk)
    p_tab, q_tab = _tc_layer2(
        parts2, inv, h1, Wn2, Ws2, b2.reshape(1, HID),
        W3[:HID], W3[HID:2 * HID], b3.reshape(1, HID),
    )

    s_arr = _sc_edge_gather(p_tab, q_tab, srcm, dstm)
    out = _tc_edge_mlp(s_arr, edge_attr.astype(f32),
                       W3[2 * HID:], W4, b4.reshape(1, 1))
    return out.reshape(-1)


# speedup vs baseline: 6.4023x; 2.9101x over previous
import functools

import jax
import jax.numpy as jnp
from jax import lax
from jax.experimental import pallas as pl
from jax.experimental.pallas import tpu as pltpu
from jax.experimental.pallas import tpu_sc as plsc

N_NODES = 10000
N_EDGES = 320000
NPAD = 10240
EPAD = 327680
D_NODE = 128
D_EDGE = 16
HID = 128

NCORES = 2
NSUB = 16
NTILES = NCORES * NSUB
CHUNK = 128
CH_PER_TILE = EPAD // NTILES // CHUNK
ROWS_PER_TILE = NPAD // NSUB

_DOT = functools.partial(
    lax.dot_general,
    precision=lax.Precision.HIGHEST,
    preferred_element_type=jnp.float32,
)


def _mm(a, b):
    return _DOT(a, b, (((1,), (0,)), ((), ())))


_sc_mesh = plsc.VectorSubcoreMesh(core_axis_name="c", subcore_axis_name="s")

_NBUF = 2
_GCHUNK = 128
_NCHUNKS = EPAD // _GCHUNK
_SLAB = 40

_SLOW_C = 1
_SLOW_SLABS = 2
_FAST_SLABS = 2
_SLOW_TOTAL = NSUB * _SLAB * _SLOW_SLABS


@functools.partial(
    pl.kernel,
    out_type=jax.ShapeDtypeStruct((NCORES, NPAD, HID), jnp.float32),
    mesh=_sc_mesh,
    scratch_types=[
        pltpu.VMEM((_SLAB, _GCHUNK), jnp.int32),
        pltpu.VMEM((_SLAB, _GCHUNK), jnp.int32),
        pltpu.VMEM((_NBUF, _GCHUNK, HID), jnp.float32),
        pltpu.VMEM_SHARED((NPAD, HID), jnp.float32),
        pltpu.SemaphoreType.DMA((_NBUF,)),
        pltpu.SemaphoreType.DMA((_NBUF,)),
    ],
)
def _sc_agg(table, srcm, dstm, zeros_hbm, parts, src_v, dst_v, rows_v, acc,
            gsem, ssem):
    c = lax.axis_index("c")
    s = lax.axis_index("s")

    for i in range(ROWS_PER_TILE // CHUNK):
        pltpu.sync_copy(
            zeros_hbm, acc.at[pl.ds(s * ROWS_PER_TILE + i * CHUNK, CHUNK)]
        )
    plsc.subcore_barrier()

    slow = c == _SLOW_C
    nslab = jnp.where(slow, _SLOW_SLABS, _FAST_SLABS)
    tile_base = jnp.where(slow, s * (_SLAB * _SLOW_SLABS),
                          _SLOW_TOTAL + s * (_SLAB * _FAST_SLABS))

    def slab_body(h, carry):
        base_ch = tile_base + h * _SLAB
        pltpu.sync_copy(srcm.at[pl.ds(base_ch, _SLAB)], src_v)
        pltpu.sync_copy(dstm.at[pl.ds(base_ch, _SLAB)], dst_v)

        for b in range(_NBUF):
            pltpu.async_copy(table.at[src_v.at[b]], rows_v.at[b], gsem.at[b])

        def group(jj, carry2):
            base = jj * _NBUF
            for b in range(_NBUF):
                j = base + b
                pltpu.make_async_copy(
                    table.at[src_v.at[j]], rows_v.at[b], gsem.at[b]
                ).wait()
                pltpu.async_copy(
                    rows_v.at[b], acc.at[dst_v.at[j]], ssem.at[b], add=True
                )
            for b in range(_NBUF):
                j = base + b
                jn = j + _NBUF
                pltpu.make_async_copy(
                    rows_v.at[b], acc.at[dst_v.at[j]], ssem.at[b]
                ).wait()

                @pl.when(jn < _SLAB)
                def _():
                    pltpu.async_copy(table.at[src_v.at[jn]], rows_v.at[b],
                                     gsem.at[b])

            return carry2

        lax.fori_loop(0, _SLAB // _NBUF, group, 0)
        return carry

    lax.fori_loop(0, nslab, slab_body, 0)

    plsc.subcore_barrier()

    pltpu.sync_copy(
        acc.at[pl.ds(s * ROWS_PER_TILE, ROWS_PER_TILE)],
        parts.at[c, pl.ds(s * ROWS_PER_TILE, ROWS_PER_TILE)],
    )


_CGRP = 8


@functools.partial(
    pl.kernel,
    out_type=jax.ShapeDtypeStruct((NCORES, NPAD, HID), jnp.float32),
    mesh=_sc_mesh,
    scratch_types=[
        pltpu.VMEM((CH_PER_TILE, CHUNK), jnp.int32),
        pltpu.VMEM((CHUNK, HID), jnp.float32),
        pltpu.VMEM_SHARED((NPAD, HID), jnp.float32),
        pltpu.SemaphoreType.DMA,
    ],
)
def _sc_count(dstm, ones_hbm, zeros_hbm, parts, dst_v, ones_v, acc, sem):
    c = lax.axis_index("c")
    s = lax.axis_index("s")
    wid = c * NSUB + s

    for i in range(ROWS_PER_TILE // CHUNK):
        pltpu.sync_copy(
            zeros_hbm, acc.at[pl.ds(s * ROWS_PER_TILE + i * CHUNK, CHUNK)]
        )
    plsc.subcore_barrier()
    pltpu.sync_copy(ones_hbm, ones_v)
    pltpu.sync_copy(dstm.at[pl.ds(wid * CH_PER_TILE, CH_PER_TILE)], dst_v)

    def group(jj, carry):
        base = jj * _CGRP
        for b in range(_CGRP):
            pltpu.async_copy(ones_v, acc.at[dst_v.at[base + b]], sem,
                             add=True)
        for b in range(_CGRP):
            pltpu.make_async_copy(ones_v, acc.at[dst_v.at[base + b]],
                                  sem).wait()
        return carry

    lax.fori_loop(0, CH_PER_TILE // _CGRP, group, 0)
    plsc.subcore_barrier()

    pltpu.sync_copy(
        acc.at[pl.ds(s * ROWS_PER_TILE, ROWS_PER_TILE)],
        parts.at[c, pl.ds(s * ROWS_PER_TILE, ROWS_PER_TILE)],
    )


@functools.partial(
    pl.kernel,
    out_type=jax.ShapeDtypeStruct((EPAD, HID), jnp.float32),
    mesh=_sc_mesh,
    scratch_types=[
        pltpu.VMEM((_SLAB * _FAST_SLABS, _GCHUNK), jnp.int32),
        pltpu.VMEM((_SLAB * _FAST_SLABS, _GCHUNK), jnp.int32),
        pltpu.VMEM((_NBUF, _GCHUNK, HID), jnp.float32),
        pltpu.VMEM((_NBUF, _GCHUNK, HID), jnp.float32),
        pltpu.SemaphoreType.DMA((_NBUF,)),
        pltpu.SemaphoreType.DMA((_NBUF,)),
        pltpu.SemaphoreType.DMA((_NBUF,)),
    ],
)
def _sc_edge_gather(ptab, qtab, srcm, dstm, s_out, src_v, dst_v, bufp, bufq,
                    psem, qsem, wsem):
    c = lax.axis_index("c")
    s = lax.axis_index("s")

    slow = c == _SLOW_C
    nslab = jnp.where(slow, _SLOW_SLABS, _FAST_SLABS)
    nch = nslab * _SLAB
    tile_base = jnp.where(slow, s * (_SLAB * _SLOW_SLABS),
                          _SLOW_TOTAL + s * (_SLAB * _FAST_SLABS))
    ebase = tile_base * _GCHUNK

    def ld_slab(h, carry):
        pltpu.sync_copy(srcm.at[pl.ds(tile_base + h * _SLAB, _SLAB)],
                        src_v.at[pl.ds(h * _SLAB, _SLAB)])
        pltpu.sync_copy(dstm.at[pl.ds(tile_base + h * _SLAB, _SLAB)],
                        dst_v.at[pl.ds(h * _SLAB, _SLAB)])
        return carry

    lax.fori_loop(0, nslab, ld_slab, 0)

    for b in range(_NBUF):
        pltpu.async_copy(ptab.at[src_v.at[b]], bufp.at[b], psem.at[b])
        pltpu.async_copy(qtab.at[dst_v.at[b]], bufq.at[b], qsem.at[b])

    def group(jj, carry):
        for b in range(_NBUF):
            j = jj * _NBUF + b
            jn = j + _NBUF
            pltpu.make_async_copy(ptab.at[src_v.at[j]], bufp.at[b],
                                  psem.at[b]).wait()
            pltpu.make_async_copy(qtab.at[dst_v.at[j]], bufq.at[b],
                                  qsem.at[b]).wait()

            def add_row(r, inner):
                for k in range(HID // 16):
                    sl = pl.ds(k * 16, 16)
                    bufp[b, r, sl] = bufp[b, r, sl] + bufq[b, r, sl]
                return inner

            lax.fori_loop(0, _GCHUNK, add_row, 0)
            pltpu.async_copy(
                bufp.at[b], s_out.at[pl.ds(ebase + j * _GCHUNK, _GCHUNK)],
                wsem.at[b]
            )

            @pl.when(jn < nch)
            def _():
                pltpu.make_async_copy(
                    bufp.at[b], s_out.at[pl.ds(ebase + j * _GCHUNK, _GCHUNK)],
                    wsem.at[b]
                ).wait()
                pltpu.async_copy(ptab.at[src_v.at[jn]], bufp.at[b],
                                 psem.at[b])
                pltpu.async_copy(qtab.at[dst_v.at[jn]], bufq.at[b],
                                 qsem.at[b])

        return carry

    lax.fori_loop(0, nch // _NBUF, group, 0)
    for i in range(_NBUF):
        j = nch - _NBUF + i
        pltpu.make_async_copy(
            bufp.at[i], s_out.at[pl.ds(ebase + j * _GCHUNK, _GCHUNK)],
            wsem.at[i]
        ).wait()


_NODE_BLK = 1024
_NODE_GRID = NPAD // _NODE_BLK


def _tc_layer1(parts, cparts, x, wn, ws, b):
    def kern(p_ref, c_ref, x_ref, wn_ref, ws_ref, b_ref, h_ref, inv_ref):
        cnt = (c_ref[0] + c_ref[1])[:, 0:1]
        inv = 1.0 / jnp.maximum(cnt, 1.0)
        mean = (p_ref[0] + p_ref[1]) * inv
        h = _mm(mean, wn_ref[...]) + _mm(x_ref[...], ws_ref[...]) + b_ref[...]
        h_ref[...] = jnp.maximum(h, 0.0)
        inv_ref[...] = inv

    return pl.pallas_call(
        kern,
        grid=(_NODE_GRID,),
        in_specs=[
            pl.BlockSpec((NCORES, _NODE_BLK, HID), lambda i: (0, i, 0)),
            pl.BlockSpec((NCORES, _NODE_BLK, HID), lambda i: (0, i, 0)),
            pl.BlockSpec((_NODE_BLK, D_NODE), lambda i: (i, 0)),
            pl.BlockSpec((D_NODE, HID), lambda i: (0, 0)),
            pl.BlockSpec((D_NODE, HID), lambda i: (0, 0)),
            pl.BlockSpec((1, HID), lambda i: (0, 0)),
        ],
        out_specs=[
            pl.BlockSpec((_NODE_BLK, HID), lambda i: (i, 0)),
            pl.BlockSpec((_NODE_BLK, 1), lambda i: (i, 0)),
        ],
        out_shape=[
            jax.ShapeDtypeStruct((NPAD, HID), jnp.float32),
            jax.ShapeDtypeStruct((NPAD, 1), jnp.float32),
        ],
    )(parts, cparts, x, wn, ws, b)


def _tc_layer2(parts, inv, h1, wn, ws, b, w3a, w3b, b3):
    def kern(p_ref, inv_ref, h1_ref, wn_ref, ws_ref, b_ref, w3a_ref, w3b_ref,
             b3_ref, p_out, q_out):
        mean = (p_ref[0] + p_ref[1]) * inv_ref[...]
        h2 = _mm(mean, wn_ref[...]) + _mm(h1_ref[...], ws_ref[...]) + b_ref[...]
        h2 = jnp.maximum(h2, 0.0)
        p_out[...] = _mm(h2, w3a_ref[...]) + b3_ref[...]
        q_out[...] = _mm(h2, w3b_ref[...])

    return pl.pallas_call(
        kern,
        grid=(_NODE_GRID,),
        in_specs=[
            pl.BlockSpec((NCORES, _NODE_BLK, HID), lambda i: (0, i, 0)),
            pl.BlockSpec((_NODE_BLK, 1), lambda i: (i, 0)),
            pl.BlockSpec((_NODE_BLK, HID), lambda i: (i, 0)),
            pl.BlockSpec((HID, HID), lambda i: (0, 0)),
            pl.BlockSpec((HID, HID), lambda i: (0, 0)),
            pl.BlockSpec((1, HID), lambda i: (0, 0)),
            pl.BlockSpec((HID, HID), lambda i: (0, 0)),
            pl.BlockSpec((HID, HID), lambda i: (0, 0)),
            pl.BlockSpec((1, HID), lambda i: (0, 0)),
        ],
        out_specs=[
            pl.BlockSpec((_NODE_BLK, HID), lambda i: (i, 0)),
            pl.BlockSpec((_NODE_BLK, HID), lambda i: (i, 0)),
        ],
        out_shape=[
            jax.ShapeDtypeStruct((NPAD, HID), jnp.float32),
            jax.ShapeDtypeStruct((NPAD, HID), jnp.float32),
        ],
    )(parts, inv, h1, wn, ws, b, w3a, w3b, b3)


_EDGE_BLK = 8192
_EDGE_GRID = EPAD // _EDGE_BLK


def _tc_edge_mlp(s_arr, attrT, w3c, w4r, b4):
    def kern(s_ref, a_ref, w3c_ref, w4_ref, b4_ref, o_ref):
        r = lax.dot_general(a_ref[...], w3c_ref[...], (((0,), (0,)), ((), ())),
                            preferred_element_type=jnp.float32)
        u = jnp.maximum(s_ref[...] + r, 0.0)
        o_ref[...] = jnp.sum(u * w4_ref[...], axis=1) + b4_ref[0, 0]

    return pl.pallas_call(
        kern,
        grid=(_EDGE_GRID,),
        in_specs=[
            pl.BlockSpec((_EDGE_BLK, HID), lambda i: (i, 0)),
            pl.BlockSpec((D_EDGE, _EDGE_BLK), lambda i: (0, i)),
            pl.BlockSpec((D_EDGE, HID), lambda i: (0, 0)),
            pl.BlockSpec((1, HID), lambda i: (0, 0)),
            pl.BlockSpec((1, 1), lambda i: (0, 0)),
        ],
        out_specs=pl.BlockSpec((_EDGE_BLK,), lambda i: (i,)),
        out_shape=jax.ShapeDtypeStruct((EPAD,), jnp.float32),
    )(s_arr, attrT, w3c, w4r, b4)


def kernel(x, edge_index, edge_attr, Wn1, Ws1, b1, Wn2, Ws2, b2, W3, b3, W4, b4):
    f32 = jnp.float32

    src = edge_index[0].astype(jnp.int32)
    dst = edge_index[1].astype(jnp.int32)
    pad = N_NODES + (jnp.arange(EPAD - N_EDGES, dtype=jnp.int32) % 128)
    srcp = jnp.concatenate([src, pad])
    dstp = jnp.concatenate([dst, pad])
    srcg = srcp.reshape(EPAD // _GCHUNK, _GCHUNK)
    dstg = dstp.reshape(EPAD // _GCHUNK, _GCHUNK)
    dstm = dstp.reshape(EPAD // CHUNK, CHUNK)

    xpad = jnp.zeros((NPAD, D_NODE), f32).at[:N_NODES].set(x.astype(f32))
    zeros_blk = jnp.zeros((CHUNK, HID), f32)
    ones_blk = jnp.ones((CHUNK, HID), f32)

    cparts = _sc_count(dstm, ones_blk, zeros_blk)

    parts1 = _sc_agg(xpad, srcg, dstg, zeros_blk)
    h1, inv = _tc_layer1(parts1, cparts, xpad, Wn1, Ws1, b1.reshape(1, HID))

    parts2 = _sc_agg(h1, srcg, dstg, zeros_blk)
    p_tab, q_tab = _tc_layer2(
        parts2, inv, h1, Wn2, Ws2, b2.reshape(1, HID),
        W3[:HID], W3[HID:2 * HID], b3.reshape(1, HID),
    )

    s_arr = _sc_edge_gather(p_tab, q_tab, srcg, dstg)
    attrT = jnp.zeros((D_EDGE, EPAD), f32).at[:, :N_EDGES].set(
        edge_attr.astype(f32).T)
    out = _tc_edge_mlp(s_arr, attrT, W3[2 * HID:], W4.reshape(1, HID),
                       b4.reshape(1, 1))
    return out[:N_EDGES]
